# baseline (speedup 1.0000x reference)
import functools

import jax
import jax.numpy as jnp
from jax import lax
from jax.experimental import pallas as pl
from jax.experimental.pallas import tpu as pltpu
from jax.experimental.pallas import tpu_sc as plsc

_LANES = 16


def _rsqrt(v):
    i = lax.bitcast_convert_type(v, jnp.int32)
    i = jnp.int32(0x5F3759DF) - (i >> 1)
    y = lax.bitcast_convert_type(i, jnp.float32)
    for _ in range(3):
        y = y * (1.5 - 0.5 * v * y * y)
    return y


def _pack_nodes_tc(feat, waug, rows_per_blk):
    n, hid = feat.shape

    def body(f_ref, w_ref, o_ref):
        f = f_ref[...]
        p = jnp.dot(f, w_ref[...], preferred_element_type=jnp.float32,
                    precision=jax.lax.Precision.HIGHEST)
        s = jnp.sum(f, axis=1, keepdims=True)
        q = jnp.sum(f * f, axis=1, keepdims=True)
        o_ref[...] = jnp.concatenate([p[:, :6], s, q], axis=1)

    return pl.pallas_call(
        body,
        grid=(n // rows_per_blk,),
        in_specs=[
            pl.BlockSpec((rows_per_blk, hid), lambda i: (i, 0)),
            pl.BlockSpec((hid, 8), lambda i: (0, 0)),
        ],
        out_specs=pl.BlockSpec((rows_per_blk, 8), lambda i: (i, 0)),
        out_shape=jax.ShapeDtypeStruct((n, 8), jnp.float32),
    )(feat, waug)


def _make_sc_kernel(nnz, hid2, ch, j_per_ch, n_chunks, iters, nw, num_cores):
    mesh = plsc.VectorSubcoreMesh(core_axis_name="c", subcore_axis_name="s")
    inv_hid2 = 1.0 / float(hid2)
    groups = ch // _LANES

    @functools.partial(
        pl.kernel,
        out_type=jax.ShapeDtypeStruct((nnz // 128, 8, 128), jnp.float32),
        mesh=mesh,
        compiler_params=pltpu.CompilerParams(
            needs_layout_passes=False, use_tc_tiling_on_sc=False),
        scratch_types=[
            pltpu.VMEM((2, j_per_ch, 2, 128), jnp.int32),
            pltpu.VMEM((2, ch, 8), jnp.float32),
            pltpu.VMEM((2, ch, 8), jnp.float32),
            pltpu.VMEM((ch // 128, 8, 128), jnp.float32),
            pltpu.VMEM((6, _LANES), jnp.float32),
            pltpu.SemaphoreType.DMA,
            pltpu.SemaphoreType.DMA,
        ],
    )
    def sc_kernel(x8, e8, idx3d, c0_hbm, out,
                  ij, xs, es, ob, c0v, sem0, sem1):
        wid = lax.axis_index("s") * num_cores + lax.axis_index("c")
        pltpu.sync_copy(c0_hbm, c0v)
        iota = lax.iota(jnp.int32, _LANES)
        c0 = [c0v[d] for d in range(6)]
        sems = (sem0, sem1)

        def fire(t, slot):
            sem = sems[slot]
            pltpu.sync_copy(
                idx3d.at[pl.ds(t * j_per_ch, j_per_ch)], ij.at[slot])
            for j in range(j_per_ch):
                pltpu.async_copy(
                    x8.at[ij.at[slot, j, 0]],
                    xs.at[slot, pl.ds(j * 128, 128)], sem)
                pltpu.async_copy(
                    e8.at[ij.at[slot, j, 1]],
                    es.at[slot, pl.ds(j * 128, 128)], sem)

        def drain(slot):
            sem = sems[slot]
            pltpu.make_async_copy(
                x8.at[pl.ds(0, ch)], xs.at[slot], sem).wait()
            pltpu.make_async_copy(
                e8.at[pl.ds(0, ch)], es.at[slot], sem).wait()

        @pl.when(wid < n_chunks)
        def _():
            fire(wid, 0)

        def step(t, slot):
            t_next = t + nw

            @pl.when(t_next < n_chunks)
            def _():
                fire(t_next, 1 - slot)

            @pl.when(t < n_chunks)
            def _():
                drain(slot)
                xsl = xs.at[slot]
                esl = es.at[slot]

                def group_body(g, gcarry):
                    k = g >> 3
                    s = g & 7
                    rows = jnp.full((_LANES,), g * _LANES, jnp.int32) + iota
                    xf = [plsc.load_gather(
                        xsl, [rows, jnp.full((_LANES,), f, jnp.int32)])
                        for f in range(8)]
                    ef = [plsc.load_gather(
                        esl, [rows, jnp.full((_LANES,), f, jnp.int32)])
                        for f in range(8)]
                    mu = (xf[6] + ef[6]) * inv_hid2
                    var = jnp.maximum((xf[7] + ef[7]) * inv_hid2 - mu * mu, 0.0)
                    inv = _rsqrt(var + 1e-5)
                    for d in range(6):
                        h = (xf[d] + ef[d]) * inv + c0[d]
                        o = 1.0 / (1.0 + jnp.exp(-h))
                        ob[k, d, pl.ds(s * _LANES, _LANES)] = o
                    return gcarry

                lax.fori_loop(0, groups, group_body, 0)
                pltpu.sync_copy(
                    ob, out.at[pl.ds(t * (ch // 128), ch // 128)])

        def pair_body(i2, carry):
            t0 = wid + (i2 * 2) * nw
            step(t0, 0)
            step(t0 + nw, 1)
            return carry

        lax.fori_loop(0, (iters + 1) // 2, pair_body, 0)

    return sc_kernel


def _detile_tc(tiles3d, nnz, k_per_blk=125):
    n_tiles = tiles3d.shape[0]

    def body(i_ref, o_ref):
        for k in range(k_per_blk):
            o_ref[:, pl.ds(k * 128, 128)] = i_ref[k, 0:6, :]

    return pl.pallas_call(
        body,
        grid=(n_tiles // k_per_blk,),
        in_specs=[pl.BlockSpec((k_per_blk, 8, 128), lambda i: (i, 0, 0))],
        out_specs=pl.BlockSpec((6, k_per_blk * 128), lambda i: (0, i)),
        out_shape=jax.ShapeDtypeStruct((6, nnz), jnp.float32),
    )(tiles3d)


def kernel(x, e, hyperedge_index, ln_scale, ln_bias, W, b):
    n_nodes, hid = x.shape
    n_hedges = e.shape[0]
    nnz = hyperedge_index.shape[1]
    hid2 = 2 * hid

    ch = 1280
    j_per_ch = ch // 128
    n_chunks = nnz // ch
    idx3d = hyperedge_index.astype(jnp.int32).reshape(
        2, nnz // 128, 128).transpose(1, 0, 2)

    ws = ln_scale[:, None] * W
    sv = ln_scale @ W
    c0 = ln_bias @ W + b
    zeros = jnp.zeros((hid, 2), jnp.float32)
    waug_x = jnp.concatenate([ws[:hid] - sv[None, :] / hid2, zeros], axis=1)
    waug_e = jnp.concatenate([ws[hid:] - sv[None, :] / hid2, zeros], axis=1)

    x8 = _pack_nodes_tc(x, waug_x, 5000)
    e8 = _pack_nodes_tc(e, waug_e, 5000)

    c0_splat = jnp.broadcast_to(c0[:, None], (6, _LANES)).astype(jnp.float32)

    info = plsc.get_sparse_core_info()
    nw = info.num_cores * info.num_subcores

    iters = -(-n_chunks // nw)

    sc = _make_sc_kernel(nnz, hid2, ch, j_per_ch, n_chunks, iters, nw,
                         info.num_cores)
    return _detile_tc(sc(x8, e8, idx3d, c0_splat), nnz).T


# speedup vs baseline: 1.0314x; 1.0314x over previous
import functools

import jax
import jax.numpy as jnp
from jax import lax
from jax.experimental import pallas as pl
from jax.experimental.pallas import tpu as pltpu
from jax.experimental.pallas import tpu_sc as plsc

_LANES = 16


def _rsqrt(v):
    i = lax.bitcast_convert_type(v, jnp.int32)
    i = jnp.int32(0x5F3759DF) - (i >> 1)
    y = lax.bitcast_convert_type(i, jnp.float32)
    for _ in range(3):
        y = y * (1.5 - 0.5 * v * y * y)
    return y


def _pack_nodes_tc(feat, waug, rows_per_blk):
    n, hid = feat.shape

    def body(f_ref, w_ref, o_ref):
        f = f_ref[...]
        p = jnp.dot(f, w_ref[...], preferred_element_type=jnp.float32,
                    precision=jax.lax.Precision.HIGHEST)
        s = jnp.sum(f, axis=1, keepdims=True)
        q = jnp.sum(f * f, axis=1, keepdims=True)
        o_ref[...] = jnp.concatenate([p[:, :6], s, q], axis=1)

    return pl.pallas_call(
        body,
        grid=(n // rows_per_blk,),
        in_specs=[
            pl.BlockSpec((rows_per_blk, hid), lambda i: (i, 0)),
            pl.BlockSpec((hid, 8), lambda i: (0, 0)),
        ],
        out_specs=pl.BlockSpec((rows_per_blk, 8), lambda i: (i, 0)),
        out_shape=jax.ShapeDtypeStruct((n, 8), jnp.float32),
    )(feat, waug)


def _make_sc_kernel(nnz, hid2, ch, j_per_ch, n_chunks, iters, nw, num_cores):
    mesh = plsc.VectorSubcoreMesh(core_axis_name="c", subcore_axis_name="s")
    inv_hid2 = 1.0 / float(hid2)
    groups = ch // _LANES

    @functools.partial(
        pl.kernel,
        out_type=jax.ShapeDtypeStruct((nnz // 128, 8, 128), jnp.float32),
        mesh=mesh,
        compiler_params=pltpu.CompilerParams(
            needs_layout_passes=False, use_tc_tiling_on_sc=False),
        scratch_types=[
            pltpu.VMEM((2, j_per_ch, 2, 128), jnp.int32),
            pltpu.VMEM((2, ch, 8), jnp.float32),
            pltpu.VMEM((2, ch, 8), jnp.float32),
            pltpu.VMEM((ch // 128, 8, 128), jnp.float32),
            pltpu.VMEM((6, _LANES), jnp.float32),
            pltpu.SemaphoreType.DMA,
            pltpu.SemaphoreType.DMA,
        ],
    )
    def sc_kernel(x8, e8, idx3d, c0_hbm, out,
                  ij, xs, es, ob, c0v, sem0, sem1):
        wid = lax.axis_index("s") * num_cores + lax.axis_index("c")
        pltpu.sync_copy(c0_hbm, c0v)
        iota = lax.iota(jnp.int32, _LANES)
        c0 = [c0v[d] for d in range(6)]
        sems = (sem0, sem1)

        def fire(t, slot):
            sem = sems[slot]
            pltpu.sync_copy(
                idx3d.at[pl.ds(t * j_per_ch, j_per_ch)], ij.at[slot])
            for j in range(j_per_ch):
                pltpu.async_copy(
                    x8.at[ij.at[slot, j, 0]],
                    xs.at[slot, pl.ds(j * 128, 128)], sem)
                pltpu.async_copy(
                    e8.at[ij.at[slot, j, 1]],
                    es.at[slot, pl.ds(j * 128, 128)], sem)

        def drain(slot):
            sem = sems[slot]
            pltpu.make_async_copy(
                x8.at[pl.ds(0, ch)], xs.at[slot], sem).wait()
            pltpu.make_async_copy(
                e8.at[pl.ds(0, ch)], es.at[slot], sem).wait()

        @pl.when(wid < n_chunks)
        def _():
            fire(wid, 0)

        def step(t, slot):
            t_next = t + nw

            @pl.when(t_next < n_chunks)
            def _():
                fire(t_next, 1 - slot)

            @pl.when(t < n_chunks)
            def _():
                drain(slot)
                xsl = xs.at[slot]
                esl = es.at[slot]

                def group_body(g, gcarry):
                    k = g >> 3
                    s = g & 7
                    rows = jnp.full((_LANES,), g * _LANES, jnp.int32) + iota
                    xf = [plsc.load_gather(
                        xsl, [rows, jnp.full((_LANES,), f, jnp.int32)])
                        for f in range(8)]
                    ef = [plsc.load_gather(
                        esl, [rows, jnp.full((_LANES,), f, jnp.int32)])
                        for f in range(8)]
                    mu = (xf[6] + ef[6]) * inv_hid2
                    var = jnp.maximum((xf[7] + ef[7]) * inv_hid2 - mu * mu, 0.0)
                    inv = _rsqrt(var + 1e-5)
                    for d in range(6):
                        h = (xf[d] + ef[d]) * inv + c0[d]
                        o = 1.0 / (1.0 + jnp.exp(-h))
                        ob[k, d, pl.ds(s * _LANES, _LANES)] = o
                    return gcarry

                lax.fori_loop(0, groups, group_body, 0)
                pltpu.sync_copy(
                    ob, out.at[pl.ds(t * (ch // 128), ch // 128)])

        def pair_body(i2, carry):
            t0 = wid + (i2 * 2) * nw
            step(t0, 0)
            step(t0 + nw, 1)
            return carry

        lax.fori_loop(0, (iters + 1) // 2, pair_body, 0)

    return sc_kernel


def _detile_tc(tiles3d, nnz, k_per_blk=250):
    n_tiles = tiles3d.shape[0]

    def body(i_ref, o_ref):
        for k in range(k_per_blk):
            o_ref[:, pl.ds(k * 128, 128)] = i_ref[k, 0:6, :]

    return pl.pallas_call(
        body,
        grid=(n_tiles // k_per_blk,),
        in_specs=[pl.BlockSpec((k_per_blk, 8, 128), lambda i: (i, 0, 0))],
        out_specs=pl.BlockSpec((6, k_per_blk * 128), lambda i: (0, i)),
        out_shape=jax.ShapeDtypeStruct((6, nnz), jnp.float32),
    )(tiles3d)


def kernel(x, e, hyperedge_index, ln_scale, ln_bias, W, b):
    n_nodes, hid = x.shape
    n_hedges = e.shape[0]
    nnz = hyperedge_index.shape[1]
    hid2 = 2 * hid

    ch = 1280
    j_per_ch = ch // 128
    n_chunks = nnz // ch
    idx3d = hyperedge_index.astype(jnp.int32).reshape(
        2, nnz // 128, 128).transpose(1, 0, 2)

    ws = ln_scale[:, None] * W
    sv = ln_scale @ W
    c0 = ln_bias @ W + b
    zeros = jnp.zeros((hid, 2), jnp.float32)
    waug_x = jnp.concatenate([ws[:hid] - sv[None, :] / hid2, zeros], axis=1)
    waug_e = jnp.concatenate([ws[hid:] - sv[None, :] / hid2, zeros], axis=1)

    x8 = _pack_nodes_tc(x, waug_x, 5000)
    e8 = _pack_nodes_tc(e, waug_e, 5000)

    c0_splat = jnp.broadcast_to(c0[:, None], (6, _LANES)).astype(jnp.float32)

    info = plsc.get_sparse_core_info()
    nw = info.num_cores * info.num_subcores

    iters = -(-n_chunks // nw)

    sc = _make_sc_kernel(nnz, hid2, ch, j_per_ch, n_chunks, iters, nw,
                         info.num_cores)
    return _detile_tc(sc(x8, e8, idx3d, c0_splat), nnz).T


# speedup vs baseline: 1.0613x; 1.0290x over previous
import functools

import jax
import jax.numpy as jnp
from jax import lax
from jax.experimental import pallas as pl
from jax.experimental.pallas import tpu as pltpu
from jax.experimental.pallas import tpu_sc as plsc

_LANES = 16


def _rsqrt(v):
    i = lax.bitcast_convert_type(v, jnp.int32)
    i = jnp.int32(0x5F3759DF) - (i >> 1)
    y = lax.bitcast_convert_type(i, jnp.float32)
    for _ in range(3):
        y = y * (1.5 - 0.5 * v * y * y)
    return y


def _pack_nodes_tc(feat, waug, rows_per_blk):
    n, hid = feat.shape

    def body(f_ref, w_ref, o_ref):
        f = f_ref[...]
        p = jnp.dot(f, w_ref[...], preferred_element_type=jnp.float32,
                    precision=jax.lax.Precision.HIGHEST)
        s = jnp.sum(f, axis=1, keepdims=True)
        q = jnp.sum(f * f, axis=1, keepdims=True)
        o_ref[...] = jnp.concatenate([p[:, :6], s, q], axis=1)

    return pl.pallas_call(
        body,
        grid=(n // rows_per_blk,),
        in_specs=[
            pl.BlockSpec((rows_per_blk, hid), lambda i: (i, 0)),
            pl.BlockSpec((hid, 8), lambda i: (0, 0)),
        ],
        out_specs=pl.BlockSpec((rows_per_blk, 8), lambda i: (i, 0)),
        out_shape=jax.ShapeDtypeStruct((n, 8), jnp.float32),
    )(feat, waug)


def _make_sc_kernel(nnz, hid2, ch, j_per_ch, n_chunks, iters, nw, num_cores):
    mesh = plsc.VectorSubcoreMesh(core_axis_name="c", subcore_axis_name="s")
    inv_hid2 = 1.0 / float(hid2)
    groups = ch // _LANES

    @functools.partial(
        pl.kernel,
        out_type=jax.ShapeDtypeStruct((nnz // 128, 8, 128), jnp.float32),
        mesh=mesh,
        compiler_params=pltpu.CompilerParams(
            needs_layout_passes=False, use_tc_tiling_on_sc=False),
        scratch_types=[
            pltpu.VMEM((2, j_per_ch, 2, 128), jnp.int32),
            pltpu.VMEM((2, ch, 8), jnp.float32),
            pltpu.VMEM((2, ch, 8), jnp.float32),
            pltpu.VMEM((ch // 128, 8, 128), jnp.float32),
            pltpu.VMEM((6, _LANES), jnp.float32),
            pltpu.SemaphoreType.DMA,
            pltpu.SemaphoreType.DMA,
        ],
    )
    def sc_kernel(x8, e8, idx3d, c0_hbm, out,
                  ij, xs, es, ob, c0v, sem0, sem1):
        wid = lax.axis_index("s") * num_cores + lax.axis_index("c")
        pltpu.sync_copy(c0_hbm, c0v)
        iota = lax.iota(jnp.int32, _LANES)
        c0 = [c0v[d] for d in range(6)]
        sems = (sem0, sem1)

        def fire(t, slot):
            sem = sems[slot]
            pltpu.sync_copy(
                idx3d.at[pl.ds(t * j_per_ch, j_per_ch)], ij.at[slot])
            for j in range(j_per_ch):
                pltpu.async_copy(
                    x8.at[ij.at[slot, j, 0]],
                    xs.at[slot, pl.ds(j * 128, 128)], sem)
                pltpu.async_copy(
                    e8.at[ij.at[slot, j, 1]],
                    es.at[slot, pl.ds(j * 128, 128)], sem)

        def drain(slot):
            sem = sems[slot]
            pltpu.make_async_copy(
                x8.at[pl.ds(0, ch)], xs.at[slot], sem).wait()
            pltpu.make_async_copy(
                e8.at[pl.ds(0, ch)], es.at[slot], sem).wait()

        @pl.when(wid < n_chunks)
        def _():
            fire(wid, 0)

        def step(t, slot):
            t_next = t + nw

            @pl.when(t_next < n_chunks)
            def _():
                fire(t_next, 1 - slot)

            @pl.when(t < n_chunks)
            def _():
                drain(slot)
                xsl = xs.at[slot]
                esl = es.at[slot]

                def group_body(g, gcarry):
                    k = g >> 3
                    s = g & 7
                    rows = jnp.full((_LANES,), g * _LANES, jnp.int32) + iota
                    xf = [plsc.load_gather(
                        xsl, [rows, jnp.full((_LANES,), f, jnp.int32)])
                        for f in range(8)]
                    ef = [plsc.load_gather(
                        esl, [rows, jnp.full((_LANES,), f, jnp.int32)])
                        for f in range(8)]
                    mu = (xf[6] + ef[6]) * inv_hid2
                    var = jnp.maximum((xf[7] + ef[7]) * inv_hid2 - mu * mu, 0.0)
                    inv = _rsqrt(var + 1e-5)
                    for d in range(6):
                        h = (xf[d] + ef[d]) * inv + c0[d]
                        o = 1.0 / (1.0 + jnp.exp(-h))
                        ob[k, d, pl.ds(s * _LANES, _LANES)] = o
                    return gcarry

                lax.fori_loop(0, groups, group_body, 0)
                pltpu.sync_copy(
                    ob, out.at[pl.ds(t * (ch // 128), ch // 128)])

        def pair_body(i2, carry):
            t0 = wid + (i2 * 2) * nw
            step(t0, 0)
            step(t0 + nw, 1)
            return carry

        lax.fori_loop(0, (iters + 1) // 2, pair_body, 0)

    return sc_kernel


def _detile_tc(tiles3d, nnz, k_per_blk=625):
    n_tiles = tiles3d.shape[0]

    def body(i_ref, o_ref):
        for k in range(k_per_blk):
            o_ref[:, pl.ds(k * 128, 128)] = i_ref[k, 0:6, :]

    return pl.pallas_call(
        body,
        grid=(n_tiles // k_per_blk,),
        in_specs=[pl.BlockSpec((k_per_blk, 8, 128), lambda i: (i, 0, 0))],
        out_specs=pl.BlockSpec((6, k_per_blk * 128), lambda i: (0, i)),
        out_shape=jax.ShapeDtypeStruct((6, nnz), jnp.float32),
    )(tiles3d)


def kernel(x, e, hyperedge_index, ln_scale, ln_bias, W, b):
    n_nodes, hid = x.shape
    n_hedges = e.shape[0]
    nnz = hyperedge_index.shape[1]
    hid2 = 2 * hid

    ch = 1280
    j_per_ch = ch // 128
    n_chunks = nnz // ch
    idx3d = hyperedge_index.astype(jnp.int32).reshape(
        2, nnz // 128, 128).transpose(1, 0, 2)

    ws = ln_scale[:, None] * W
    sv = ln_scale @ W
    c0 = ln_bias @ W + b
    zeros = jnp.zeros((hid, 2), jnp.float32)
    waug_x = jnp.concatenate([ws[:hid] - sv[None, :] / hid2, zeros], axis=1)
    waug_e = jnp.concatenate([ws[hid:] - sv[None, :] / hid2, zeros], axis=1)

    x8 = _pack_nodes_tc(x, waug_x, 5000)
    e8 = _pack_nodes_tc(e, waug_e, 5000)

    c0_splat = jnp.broadcast_to(c0[:, None], (6, _LANES)).astype(jnp.float32)

    info = plsc.get_sparse_core_info()
    nw = info.num_cores * info.num_subcores

    iters = -(-n_chunks // nw)

    sc = _make_sc_kernel(nnz, hid2, ch, j_per_ch, n_chunks, iters, nw,
                         info.num_cores)
    return _detile_tc(sc(x8, e8, idx3d, c0_splat), nnz).T


# speedup vs baseline: 1.0653x; 1.0038x over previous
import functools

import jax
import jax.numpy as jnp
from jax import lax
from jax.experimental import pallas as pl
from jax.experimental.pallas import tpu as pltpu
from jax.experimental.pallas import tpu_sc as plsc

_LANES = 16


def _rsqrt(v):
    i = lax.bitcast_convert_type(v, jnp.int32)
    i = jnp.int32(0x5F3759DF) - (i >> 1)
    y = lax.bitcast_convert_type(i, jnp.float32)
    for _ in range(3):
        y = y * (1.5 - 0.5 * v * y * y)
    return y


def _pack_nodes_tc(feat, waug, rows_per_blk):
    n, hid = feat.shape

    def body(f_ref, w_ref, o_ref):
        f = f_ref[...]
        p = jnp.dot(f, w_ref[...], preferred_element_type=jnp.float32,
                    precision=jax.lax.Precision.HIGHEST)
        s = jnp.sum(f, axis=1, keepdims=True)
        q = jnp.sum(f * f, axis=1, keepdims=True)
        o_ref[...] = jnp.concatenate([p[:, :6], s, q], axis=1)

    return pl.pallas_call(
        body,
        grid=(n // rows_per_blk,),
        in_specs=[
            pl.BlockSpec((rows_per_blk, hid), lambda i: (i, 0)),
            pl.BlockSpec((hid, 8), lambda i: (0, 0)),
        ],
        out_specs=pl.BlockSpec((rows_per_blk, 8), lambda i: (i, 0)),
        out_shape=jax.ShapeDtypeStruct((n, 8), jnp.float32),
    )(feat, waug)


def _make_sc_kernel(nnz, hid2, ch, j_per_ch, n_chunks, iters, nw, num_cores):
    mesh = plsc.VectorSubcoreMesh(core_axis_name="c", subcore_axis_name="s")
    inv_hid2 = 1.0 / float(hid2)
    groups = ch // _LANES

    @functools.partial(
        pl.kernel,
        out_type=jax.ShapeDtypeStruct((nnz // 128, 8, 128), jnp.float32),
        mesh=mesh,
        compiler_params=pltpu.CompilerParams(
            needs_layout_passes=False, use_tc_tiling_on_sc=False),
        scratch_types=[
            pltpu.VMEM((2, j_per_ch, 2, 128), jnp.int32),
            pltpu.VMEM((2, ch, 8), jnp.float32),
            pltpu.VMEM((2, ch, 8), jnp.float32),
            pltpu.VMEM((ch // 128, 8, 128), jnp.float32),
            pltpu.VMEM((6, _LANES), jnp.float32),
            pltpu.SemaphoreType.DMA,
            pltpu.SemaphoreType.DMA,
        ],
    )
    def sc_kernel(x8, e8, idx3d, c0_hbm, out,
                  ij, xs, es, ob, c0v, sem0, sem1):
        wid = lax.axis_index("s") * num_cores + lax.axis_index("c")
        pltpu.sync_copy(c0_hbm, c0v)
        iota = lax.iota(jnp.int32, _LANES)
        c0 = [c0v[d] for d in range(6)]
        sems = (sem0, sem1)

        def fire(t, slot):
            sem = sems[slot]
            pltpu.sync_copy(
                idx3d.at[pl.ds(t * j_per_ch, j_per_ch)], ij.at[slot])
            for j in range(j_per_ch):
                pltpu.async_copy(
                    x8.at[ij.at[slot, j, 0]],
                    xs.at[slot, pl.ds(j * 128, 128)], sem)
                pltpu.async_copy(
                    e8.at[ij.at[slot, j, 1]],
                    es.at[slot, pl.ds(j * 128, 128)], sem)

        def drain(slot):
            sem = sems[slot]
            pltpu.make_async_copy(
                x8.at[pl.ds(0, ch)], xs.at[slot], sem).wait()
            pltpu.make_async_copy(
                e8.at[pl.ds(0, ch)], es.at[slot], sem).wait()

        @pl.when(wid < n_chunks)
        def _():
            fire(wid, 0)

        def step(t, slot):
            t_next = t + nw

            @pl.when(t_next < n_chunks)
            def _():
                fire(t_next, 1 - slot)

            @pl.when(t < n_chunks)
            def _():
                drain(slot)
                xsl = xs.at[slot]
                esl = es.at[slot]

                def group_body(g, gcarry):
                    k = g >> 3
                    s = g & 7
                    rows = jnp.full((_LANES,), g * _LANES, jnp.int32) + iota
                    xf = [plsc.load_gather(
                        xsl, [rows, jnp.full((_LANES,), f, jnp.int32)])
                        for f in range(8)]
                    ef = [plsc.load_gather(
                        esl, [rows, jnp.full((_LANES,), f, jnp.int32)])
                        for f in range(8)]
                    mu = (xf[6] + ef[6]) * inv_hid2
                    var = jnp.maximum((xf[7] + ef[7]) * inv_hid2 - mu * mu, 0.0)
                    inv = _rsqrt(var + 1e-5)
                    for d in range(6):
                        h = (xf[d] + ef[d]) * inv + c0[d]
                        o = 1.0 / (1.0 + jnp.exp(-h))
                        ob[k, d, pl.ds(s * _LANES, _LANES)] = o
                    return gcarry

                lax.fori_loop(0, groups, group_body, 0)
                pltpu.sync_copy(
                    ob, out.at[pl.ds(t * (ch // 128), ch // 128)])

        def pair_body(i2, carry):
            t0 = wid + (i2 * 2) * nw
            step(t0, 0)
            step(t0 + nw, 1)
            return carry

        lax.fori_loop(0, (iters + 1) // 2, pair_body, 0)

    return sc_kernel


def _detile_tc(tiles3d, nnz, k_per_blk=1250):
    n_tiles = tiles3d.shape[0]

    def body(i_ref, o_ref):
        for k in range(k_per_blk):
            o_ref[:, pl.ds(k * 128, 128)] = i_ref[k, 0:6, :]

    return pl.pallas_call(
        body,
        grid=(n_tiles // k_per_blk,),
        in_specs=[pl.BlockSpec((k_per_blk, 8, 128), lambda i: (i, 0, 0))],
        out_specs=pl.BlockSpec((6, k_per_blk * 128), lambda i: (0, i)),
        out_shape=jax.ShapeDtypeStruct((6, nnz), jnp.float32),
    )(tiles3d)


def kernel(x, e, hyperedge_index, ln_scale, ln_bias, W, b):
    n_nodes, hid = x.shape
    n_hedges = e.shape[0]
    nnz = hyperedge_index.shape[1]
    hid2 = 2 * hid

    ch = 1280
    j_per_ch = ch // 128
    n_chunks = nnz // ch
    idx3d = hyperedge_index.astype(jnp.int32).reshape(
        2, nnz // 128, 128).transpose(1, 0, 2)

    ws = ln_scale[:, None] * W
    sv = ln_scale @ W
    c0 = ln_bias @ W + b
    zeros = jnp.zeros((hid, 2), jnp.float32)
    waug_x = jnp.concatenate([ws[:hid] - sv[None, :] / hid2, zeros], axis=1)
    waug_e = jnp.concatenate([ws[hid:] - sv[None, :] / hid2, zeros], axis=1)

    x8 = _pack_nodes_tc(x, waug_x, 5000)
    e8 = _pack_nodes_tc(e, waug_e, 5000)

    c0_splat = jnp.broadcast_to(c0[:, None], (6, _LANES)).astype(jnp.float32)

    info = plsc.get_sparse_core_info()
    nw = info.num_cores * info.num_subcores

    iters = -(-n_chunks // nw)

    sc = _make_sc_kernel(nnz, hid2, ch, j_per_ch, n_chunks, iters, nw,
                         info.num_cores)
    return _detile_tc(sc(x8, e8, idx3d, c0_splat), nnz).T
